# baseline (speedup 1.0000x reference)
import functools

import jax
import jax.numpy as jnp
from jax import lax
from jax.experimental import pallas as pl
from jax.experimental.pallas import tpu as pltpu
from jax.experimental.pallas import tpu_sc as plsc

B = 4096
L = 200
EMBED = 64
HIDDEN = 128
OUT = 2

NC = 2
NS = 16
NW = NC * NS
ROWS_PER_W = B // NW
HALF = L // 2


def _pool_body(x_hbm, table_hbm, out_hbm, idx_v, buf0, buf1, stage, sem):
    wid = lax.axis_index("s") * NC + lax.axis_index("c")
    base = wid * ROWS_PER_W

    pltpu.sync_copy(x_hbm.at[pl.ds(base * 2, 2 * ROWS_PER_W)], idx_v)

    inv_l = jnp.float32(1.0 / L)

    def row_step(i, carry):
        cp0 = pltpu.async_copy(table_hbm.at[idx_v.at[2 * i]], buf0, sem)
        cp1 = pltpu.async_copy(table_hbm.at[idx_v.at[2 * i + 1]], buf1, sem)
        cp0.wait()
        cp1.wait()

        def red(j, accs):
            a0, a1, a2, a3 = accs
            a0 = a0 + buf0[j, pl.ds(0, 16)] + buf1[j, pl.ds(0, 16)]
            a1 = a1 + buf0[j, pl.ds(16, 16)] + buf1[j, pl.ds(16, 16)]
            a2 = a2 + buf0[j, pl.ds(32, 16)] + buf1[j, pl.ds(32, 16)]
            a3 = a3 + buf0[j, pl.ds(48, 16)] + buf1[j, pl.ds(48, 16)]
            return (a0, a1, a2, a3)

        z = jnp.zeros((16,), jnp.float32)
        a0, a1, a2, a3 = lax.fori_loop(0, HALF, red, (z, z, z, z))
        stage[i, pl.ds(0, 16)] = a0 * inv_l
        stage[i, pl.ds(16, 16)] = a1 * inv_l
        stage[i, pl.ds(32, 16)] = a2 * inv_l
        stage[i, pl.ds(48, 16)] = a3 * inv_l
        return carry

    lax.fori_loop(0, ROWS_PER_W, row_step, 0)
    pltpu.sync_copy(stage, out_hbm.at[pl.ds(base, ROWS_PER_W)])


_pool = pl.kernel(
    _pool_body,
    out_type=jax.ShapeDtypeStruct((B, EMBED), jnp.float32),
    mesh=plsc.VectorSubcoreMesh(
        core_axis_name="c", subcore_axis_name="s",
        num_cores=NC, num_subcores=NS),
    scratch_types=[
        pltpu.VMEM((2 * ROWS_PER_W, HALF), jnp.int32),
        pltpu.VMEM((HALF, EMBED), jnp.float32),
        pltpu.VMEM((HALF, EMBED), jnp.float32),
        pltpu.VMEM((ROWS_PER_W, EMBED), jnp.float32),
        pltpu.SemaphoreType.DMA,
    ],
    compiler_params=pltpu.CompilerParams(use_tc_tiling_on_sc=False),
)


def _mlp_body(p_ref, w1_ref, b1_ref, w2_ref, b2_ref, o_ref):
    h = jnp.dot(p_ref[...], w1_ref[...], preferred_element_type=jnp.float32)
    h = jnp.maximum(h + b1_ref[...], 0.0)
    o_ref[...] = (
        jnp.dot(h, w2_ref[...], preferred_element_type=jnp.float32)
        + b2_ref[...])


_MLP_BLK = 512


def _mlp(pooled, W1, b1, W2, b2):
    return pl.pallas_call(
        _mlp_body,
        grid=(B // _MLP_BLK,),
        in_specs=[
            pl.BlockSpec((_MLP_BLK, EMBED), lambda i: (i, 0)),
            pl.BlockSpec((EMBED, HIDDEN), lambda i: (0, 0)),
            pl.BlockSpec((1, HIDDEN), lambda i: (0, 0)),
            pl.BlockSpec((HIDDEN, OUT), lambda i: (0, 0)),
            pl.BlockSpec((1, OUT), lambda i: (0, 0)),
        ],
        out_specs=pl.BlockSpec((_MLP_BLK, OUT), lambda i: (i, 0)),
        out_shape=jax.ShapeDtypeStruct((B, OUT), jnp.float32),
    )(pooled, W1, b1, W2, b2)


@jax.jit
def kernel(x, table, W1, b1, W2, b2):
    x2 = x.reshape(B * 2, HALF)
    pooled = _pool(x2, table)
    return _mlp(pooled, W1, b1.reshape(1, HIDDEN), W2, b2.reshape(1, OUT))


# speedup vs baseline: 1.1366x; 1.1366x over previous
import jax
import jax.numpy as jnp
from jax import lax
from jax.experimental import pallas as pl
from jax.experimental.pallas import tpu as pltpu
from jax.experimental.pallas import tpu_sc as plsc

B = 4096
L = 200
EMBED = 64
HIDDEN = 128
OUT = 2

NC = 2
NS = 16
NW = NC * NS
ROWS_PER_W = B // NW
IDX_PER_W = ROWS_PER_W * L
SPLIT_A = 104
SPLIT_B = L - SPLIT_A


def _pool_body(x_hbm, table_hbm, out_hbm, idx_v, buf0, buf1, stage,
               sem0, sem1):
    wid = lax.axis_index("s") * NC + lax.axis_index("c")
    base = wid * ROWS_PER_W

    pltpu.sync_copy(x_hbm.at[pl.ds(base * L, IDX_PER_W)], idx_v)

    bufs = (buf0, buf1)
    sems = (sem0, sem1)
    inv_l = jnp.float32(1.0 / L)

    def issue(i, buf, sem):
        off = pl.multiple_of(i * L, 8)
        pltpu.async_copy(
            table_hbm.at[idx_v.at[pl.ds(off, SPLIT_A)]],
            buf.at[pl.ds(0, SPLIT_A)], sem)
        off2 = pl.multiple_of(i * L + SPLIT_A, 8)
        pltpu.async_copy(
            table_hbm.at[idx_v.at[pl.ds(off2, SPLIT_B)]],
            buf.at[pl.ds(SPLIT_A, SPLIT_B)], sem)

    def drain(buf, sem):
        pltpu.make_async_copy(table_hbm.at[pl.ds(0, L)], buf, sem).wait()

    def reduce_into(buf, i):
        def red(j, accs):
            a0, a1, a2, a3 = accs
            for dj in range(4):
                r = j * 4 + dj
                a0 = a0 + buf[r, pl.ds(0, 16)]
                a1 = a1 + buf[r, pl.ds(16, 16)]
                a2 = a2 + buf[r, pl.ds(32, 16)]
                a3 = a3 + buf[r, pl.ds(48, 16)]
            return (a0, a1, a2, a3)

        z = jnp.zeros((16,), jnp.float32)
        a0, a1, a2, a3 = lax.fori_loop(0, L // 4, red, (z, z, z, z))
        stage[i, pl.ds(0, 16)] = a0 * inv_l
        stage[i, pl.ds(16, 16)] = a1 * inv_l
        stage[i, pl.ds(32, 16)] = a2 * inv_l
        stage[i, pl.ds(48, 16)] = a3 * inv_l

    issue(0, bufs[0], sems[0])

    def outer(i2, carry):
        for b in range(2):
            i = i2 * 2 + b

            @pl.when(i + 1 < ROWS_PER_W)
            def _():
                issue(i + 1, bufs[1 - b], sems[1 - b])

            drain(bufs[b], sems[b])
            reduce_into(bufs[b], i)
        return carry

    lax.fori_loop(0, ROWS_PER_W // 2, outer, 0)
    pltpu.sync_copy(stage, out_hbm.at[pl.ds(base, ROWS_PER_W)])


_pool = pl.kernel(
    _pool_body,
    out_type=jax.ShapeDtypeStruct((B, EMBED), jnp.float32),
    mesh=plsc.VectorSubcoreMesh(
        core_axis_name="c", subcore_axis_name="s",
        num_cores=NC, num_subcores=NS),
    scratch_types=[
        pltpu.VMEM((IDX_PER_W,), jnp.int32),
        pltpu.VMEM((L, EMBED), jnp.float32),
        pltpu.VMEM((L, EMBED), jnp.float32),
        pltpu.VMEM((ROWS_PER_W, EMBED), jnp.float32),
        pltpu.SemaphoreType.DMA,
        pltpu.SemaphoreType.DMA,
    ],
    compiler_params=pltpu.CompilerParams(use_tc_tiling_on_sc=False),
)


def _mlp_body(p_ref, w1_ref, b1_ref, w2_ref, b2_ref, o_ref):
    h = jnp.dot(p_ref[...], w1_ref[...], preferred_element_type=jnp.float32)
    h = jnp.maximum(h + b1_ref[...], 0.0)
    o_ref[...] = (
        jnp.dot(h, w2_ref[...], preferred_element_type=jnp.float32)
        + b2_ref[...])


_MLP_BLK = 512


def _mlp(pooled, W1, b1, W2, b2):
    return pl.pallas_call(
        _mlp_body,
        grid=(B // _MLP_BLK,),
        in_specs=[
            pl.BlockSpec((_MLP_BLK, EMBED), lambda i: (i, 0)),
            pl.BlockSpec((EMBED, HIDDEN), lambda i: (0, 0)),
            pl.BlockSpec((1, HIDDEN), lambda i: (0, 0)),
            pl.BlockSpec((HIDDEN, OUT), lambda i: (0, 0)),
            pl.BlockSpec((1, OUT), lambda i: (0, 0)),
        ],
        out_specs=pl.BlockSpec((_MLP_BLK, OUT), lambda i: (i, 0)),
        out_shape=jax.ShapeDtypeStruct((B, OUT), jnp.float32),
    )(pooled, W1, b1, W2, b2)


@jax.jit
def kernel(x, table, W1, b1, W2, b2):
    x1 = x.reshape(B * L)
    pooled = _pool(x1, table)
    return _mlp(pooled, W1, b1.reshape(1, HIDDEN), W2, b2.reshape(1, OUT))


# speedup vs baseline: 1.8477x; 1.6256x over previous
import jax
import jax.numpy as jnp
from jax import lax
from jax.experimental import pallas as pl
from jax.experimental.pallas import tpu as pltpu
from jax.experimental.pallas import tpu_sc as plsc

B = 4096
L = 200
EMBED = 64
HIDDEN = 128
OUT = 2

NC = 2
NS = 16
NW = NC * NS
ROWS_PER_W = B // NW
IDX_PER_W = ROWS_PER_W * L
SPLIT_A = 104
SPLIT_B = L - SPLIT_A
NLG = HIDDEN // 16


def _pool_body(x_hbm, tw_hbm, out_hbm, idx_v, buf0, buf1, stage,
               sem0, sem1):
    wid = lax.axis_index("s") * NC + lax.axis_index("c")
    base = wid * ROWS_PER_W

    pltpu.sync_copy(x_hbm.at[pl.ds(base * L, IDX_PER_W)], idx_v)

    bufs = (buf0, buf1)
    sems = (sem0, sem1)
    inv_l = jnp.float32(1.0 / L)

    def issue(i, buf, sem):
        off = pl.multiple_of(i * L, 8)
        pltpu.async_copy(
            tw_hbm.at[idx_v.at[pl.ds(off, SPLIT_A)]],
            buf.at[pl.ds(0, SPLIT_A)], sem)
        off2 = pl.multiple_of(i * L + SPLIT_A, 8)
        pltpu.async_copy(
            tw_hbm.at[idx_v.at[pl.ds(off2, SPLIT_B)]],
            buf.at[pl.ds(SPLIT_A, SPLIT_B)], sem)

    def drain(buf, sem):
        pltpu.make_async_copy(tw_hbm.at[pl.ds(0, L)], buf, sem).wait()

    def reduce_into(buf, i):
        def red(j, accs):
            accs = list(accs)
            for dj in range(4):
                r = j * 4 + dj
                for c in range(NLG):
                    accs[c] = accs[c] + buf[r, pl.ds(c * 16, 16)]
            return tuple(accs)

        z = jnp.zeros((16,), jnp.float32)
        accs = lax.fori_loop(0, L // 4, red, (z,) * NLG)
        for c in range(NLG):
            stage[i, pl.ds(c * 16, 16)] = accs[c] * inv_l

    issue(0, bufs[0], sems[0])

    def outer(i2, carry):
        for b in range(2):
            i = i2 * 2 + b

            @pl.when(i + 1 < ROWS_PER_W)
            def _():
                issue(i + 1, bufs[1 - b], sems[1 - b])

            drain(bufs[b], sems[b])
            reduce_into(bufs[b], i)
        return carry

    lax.fori_loop(0, ROWS_PER_W // 2, outer, 0)
    pltpu.sync_copy(stage, out_hbm.at[pl.ds(base, ROWS_PER_W)])


_pool = pl.kernel(
    _pool_body,
    out_type=jax.ShapeDtypeStruct((B, HIDDEN), jnp.float32),
    mesh=plsc.VectorSubcoreMesh(
        core_axis_name="c", subcore_axis_name="s",
        num_cores=NC, num_subcores=NS),
    scratch_types=[
        pltpu.VMEM((IDX_PER_W,), jnp.int32),
        pltpu.VMEM((L, HIDDEN), jnp.float32),
        pltpu.VMEM((L, HIDDEN), jnp.float32),
        pltpu.VMEM((ROWS_PER_W, HIDDEN), jnp.float32),
        pltpu.SemaphoreType.DMA,
        pltpu.SemaphoreType.DMA,
    ],
    compiler_params=pltpu.CompilerParams(use_tc_tiling_on_sc=False),
)


def _head_body(p_ref, b1_ref, w2_ref, b2_ref, o_ref):
    h = jnp.maximum(p_ref[...] + b1_ref[...], 0.0)
    o_ref[...] = (
        jnp.dot(h, w2_ref[...], preferred_element_type=jnp.float32)
        + b2_ref[...])


_HEAD_BLK = 512


def _head(pooled, b1, W2, b2):
    return pl.pallas_call(
        _head_body,
        grid=(B // _HEAD_BLK,),
        in_specs=[
            pl.BlockSpec((_HEAD_BLK, HIDDEN), lambda i: (i, 0)),
            pl.BlockSpec((1, HIDDEN), lambda i: (0, 0)),
            pl.BlockSpec((HIDDEN, OUT), lambda i: (0, 0)),
            pl.BlockSpec((1, OUT), lambda i: (0, 0)),
        ],
        out_specs=pl.BlockSpec((_HEAD_BLK, OUT), lambda i: (i, 0)),
        out_shape=jax.ShapeDtypeStruct((B, OUT), jnp.float32),
    )(pooled, b1, W2, b2)


@jax.jit
def kernel(x, table, W1, b1, W2, b2):
    tw = jnp.dot(table, W1, preferred_element_type=jnp.float32)
    x1 = x.reshape(B * L)
    pooled = _pool(x1, tw)
    return _head(pooled, b1.reshape(1, HIDDEN), W2, b2.reshape(1, OUT))


# speedup vs baseline: 2.0149x; 1.0905x over previous
import jax
import jax.numpy as jnp
from jax import lax
from jax.experimental import pallas as pl
from jax.experimental.pallas import tpu as pltpu
from jax.experimental.pallas import tpu_sc as plsc

B = 4096
L = 200
EMBED = 64
HIDDEN = 128
OUT = 2

NC = 2
NS = 16
NW = NC * NS
ROWS_PER_W = B // NW
IDX_PER_W = ROWS_PER_W * L
SPLIT_A = 104
SPLIT_B = L - SPLIT_A
NLG = HIDDEN // 16


def _pool_body(x_hbm, tw_hbm, out_hbm, idx_v, buf0, buf1, buf2, stage,
               sem0, sem1, sem2):
    wid = lax.axis_index("s") * NC + lax.axis_index("c")
    base = wid * ROWS_PER_W

    pltpu.sync_copy(x_hbm.at[pl.ds(base * L, IDX_PER_W)], idx_v)

    bufs = (buf0, buf1, buf2)
    sems = (sem0, sem1, sem2)
    inv_l = jnp.float32(1.0 / L)

    def issue(i, buf, sem):
        off = pl.multiple_of(i * L, 8)
        pltpu.async_copy(
            tw_hbm.at[idx_v.at[pl.ds(off, SPLIT_A)]],
            buf.at[pl.ds(0, SPLIT_A)], sem)
        off2 = pl.multiple_of(i * L + SPLIT_A, 8)
        pltpu.async_copy(
            tw_hbm.at[idx_v.at[pl.ds(off2, SPLIT_B)]],
            buf.at[pl.ds(SPLIT_A, SPLIT_B)], sem)

    def drain(buf, sem):
        pltpu.make_async_copy(tw_hbm.at[pl.ds(0, L)], buf, sem).wait()

    def reduce_into(buf, i):
        def red(j, accs):
            accs = list(accs)
            for dj in range(8):
                r = j * 8 + dj
                for c in range(NLG):
                    accs[c] = accs[c] + buf[r, pl.ds(c * 16, 16)]
            return tuple(accs)

        z = jnp.zeros((16,), jnp.float32)
        accs = lax.fori_loop(0, L // 8, red, (z,) * NLG)
        for c in range(NLG):
            stage[i, pl.ds(c * 16, 16)] = accs[c] * inv_l

    issue(0, bufs[0], sems[0])
    issue(1, bufs[1], sems[1])

    def outer(i3, carry):
        for k in range(3):
            i = i3 * 3 + k

            @pl.when(i + 2 < ROWS_PER_W)
            def _():
                issue(i + 2, bufs[(k + 2) % 3], sems[(k + 2) % 3])

            drain(bufs[k], sems[k])
            reduce_into(bufs[k], i)
        return carry

    nfull = ROWS_PER_W // 3
    lax.fori_loop(0, nfull, outer, 0)
    for i in range(nfull * 3, ROWS_PER_W):
        k = i % 3
        drain(bufs[k], sems[k])
        reduce_into(bufs[k], i)
    pltpu.sync_copy(stage, out_hbm.at[pl.ds(base, ROWS_PER_W)])


_pool = pl.kernel(
    _pool_body,
    out_type=jax.ShapeDtypeStruct((B, HIDDEN), jnp.float32),
    mesh=plsc.VectorSubcoreMesh(
        core_axis_name="c", subcore_axis_name="s",
        num_cores=NC, num_subcores=NS),
    scratch_types=[
        pltpu.VMEM((IDX_PER_W,), jnp.int32),
        pltpu.VMEM((L, HIDDEN), jnp.float32),
        pltpu.VMEM((L, HIDDEN), jnp.float32),
        pltpu.VMEM((L, HIDDEN), jnp.float32),
        pltpu.VMEM((ROWS_PER_W, HIDDEN), jnp.float32),
        pltpu.SemaphoreType.DMA,
        pltpu.SemaphoreType.DMA,
        pltpu.SemaphoreType.DMA,
    ],
    compiler_params=pltpu.CompilerParams(use_tc_tiling_on_sc=False),
)


def _head_body(p_ref, b1_ref, w2_ref, b2_ref, o_ref):
    h = jnp.maximum(p_ref[...] + b1_ref[...], 0.0)
    o_ref[...] = (
        jnp.dot(h, w2_ref[...], preferred_element_type=jnp.float32)
        + b2_ref[...])


_HEAD_BLK = 512


def _head(pooled, b1, W2, b2):
    return pl.pallas_call(
        _head_body,
        grid=(B // _HEAD_BLK,),
        in_specs=[
            pl.BlockSpec((_HEAD_BLK, HIDDEN), lambda i: (i, 0)),
            pl.BlockSpec((1, HIDDEN), lambda i: (0, 0)),
            pl.BlockSpec((HIDDEN, OUT), lambda i: (0, 0)),
            pl.BlockSpec((1, OUT), lambda i: (0, 0)),
        ],
        out_specs=pl.BlockSpec((_HEAD_BLK, OUT), lambda i: (i, 0)),
        out_shape=jax.ShapeDtypeStruct((B, OUT), jnp.float32),
    )(pooled, b1, W2, b2)


@jax.jit
def kernel(x, table, W1, b1, W2, b2):
    tw = jnp.dot(table, W1, preferred_element_type=jnp.float32)
    x1 = x.reshape(B * L)
    pooled = _pool(x1, tw)
    return _head(pooled, b1.reshape(1, HIDDEN), W2, b2.reshape(1, OUT))
